# speedup vs baseline: 1.2525x; 1.2525x over previous
import jax
import jax.numpy as jnp
from jax.experimental import pallas as pl
from jax.experimental.pallas import tpu as pltpu

NUM_EXPERTS = 8
TOP_K = 2
D_MODEL = 2048
D_FF = 1024
SEQ = 2048

TBLK = 512
NT = SEQ // TBLK


def _moe_body(x_ref, gw_ref, wg_ref, wu_ref, wd_ref, out_ref, logits_ref,
              c_scr):
    e = pl.program_id(1)

    @pl.when(e == 0)
    def _router():
        x = x_ref[...]
        logits = jax.lax.dot_general(
            x, gw_ref[...], (((1,), (1,)), ((), ())),
            preferred_element_type=jnp.float32)
        logits_ref[...] = logits
        idx = jax.lax.broadcasted_iota(jnp.int32, logits.shape, 1)
        m1 = jnp.max(logits, axis=-1, keepdims=True)
        e1 = jnp.min(jnp.where(logits == m1, idx, NUM_EXPERTS), axis=-1,
                     keepdims=True)
        lm = jnp.where(idx == e1, -jnp.inf, logits)
        m2 = jnp.max(lm, axis=-1, keepdims=True)
        e2 = jnp.min(jnp.where(lm == m2, idx, NUM_EXPERTS), axis=-1,
                     keepdims=True)
        w1 = 1.0 / (1.0 + jnp.exp(m2 - m1))
        w2 = 1.0 - w1
        c_scr[...] = jnp.where(idx == e1, w1, 0.0) + jnp.where(idx == e2, w2, 0.0)

    xb = x_ref[...].astype(jnp.bfloat16)
    wg = wg_ref[0]
    wu = wu_ref[0]
    wd = wd_ref[0]
    g = jax.lax.dot_general(xb, wg, (((1,), (1,)), ((), ())),
                            preferred_element_type=jnp.float32)
    u = jax.lax.dot_general(xb, wu, (((1,), (1,)), ((), ())),
                            preferred_element_type=jnp.float32)
    h = (g * jax.nn.sigmoid(g) * u).astype(jnp.bfloat16)
    y = jax.lax.dot_general(h, wd, (((1,), (1,)), ((), ())),
                            preferred_element_type=jnp.float32)
    idx = jax.lax.broadcasted_iota(jnp.int32, (TBLK, NUM_EXPERTS), 1)
    ce = jnp.sum(jnp.where(idx == e, c_scr[...], 0.0), axis=-1, keepdims=True)
    contrib = y * ce

    @pl.when(e == 0)
    def _init():
        out_ref[...] = contrib

    @pl.when(e > 0)
    def _acc():
        out_ref[...] += contrib


def kernel(hidden_states, gate_w, w_gate, w_up, w_down):
    B, S, H = hidden_states.shape
    x = hidden_states.reshape(S, H)
    wg = w_gate.astype(jnp.bfloat16)
    wu = w_up.astype(jnp.bfloat16)
    wd = w_down.astype(jnp.bfloat16)

    out, logits = pl.pallas_call(
        _moe_body,
        grid=(NT, NUM_EXPERTS),
        in_specs=[
            pl.BlockSpec((TBLK, D_MODEL), lambda t, e: (t, 0)),
            pl.BlockSpec((NUM_EXPERTS, D_MODEL), lambda t, e: (0, 0)),
            pl.BlockSpec((1, D_FF, D_MODEL), lambda t, e: (e, 0, 0)),
            pl.BlockSpec((1, D_FF, D_MODEL), lambda t, e: (e, 0, 0)),
            pl.BlockSpec((1, D_MODEL, D_FF), lambda t, e: (e, 0, 0)),
        ],
        out_specs=[
            pl.BlockSpec((TBLK, D_MODEL), lambda t, e: (t, 0)),
            pl.BlockSpec((TBLK, NUM_EXPERTS), lambda t, e: (t, 0)),
        ],
        out_shape=[
            jax.ShapeDtypeStruct((S, D_MODEL), jnp.float32),
            jax.ShapeDtypeStruct((S, NUM_EXPERTS), jnp.float32),
        ],
        scratch_shapes=[pltpu.VMEM((TBLK, NUM_EXPERTS), jnp.float32)],
        compiler_params=pltpu.CompilerParams(
            dimension_semantics=("arbitrary", "arbitrary")),
    )(x, gate_w, wg, wu, wd)
    return out.reshape(B, S, H), logits
